# baseline (speedup 1.0000x reference)
import functools

import jax
import jax.numpy as jnp
from jax import lax
from jax.experimental import pallas as pl
from jax.experimental.pallas import tpu as pltpu
from jax.experimental.pallas import tpu_sc as plsc

_LANES = 16
_PE_ROWS = 16
_X_ROWS = 32


@functools.cache
def _make_sc_add(batch, seq_len, d_model, n_workers, n_cores):
    n_rows = batch * seq_len
    s_w = seq_len // n_workers
    n_sc = s_w // _X_ROWS
    n_steps = n_sc * batch
    n_pe = s_w // _PE_ROWS
    hgroups = _PE_ROWS * d_model // _LANES
    gpr = d_model // _LANES

    mesh = plsc.VectorSubcoreMesh(core_axis_name="c", subcore_axis_name="s")

    @functools.partial(
        pl.kernel,
        out_type=jax.ShapeDtypeStruct((n_rows, d_model), jnp.float32),
        mesh=mesh,
        scratch_types=(
            [pltpu.VMEM((2 * _X_ROWS, d_model), jnp.float32),
             pltpu.VMEM((3 * _PE_ROWS, d_model), jnp.float32)]
            + [pltpu.SemaphoreType.DMA for _ in range(3)]
        ),
        compiler_params=pltpu.CompilerParams(use_tc_tiling_on_sc=True),
    )
    def sc_add(x_hbm, pe_hbm, o_hbm, obuf, pbuf, sem_x, sem_o, sem_p):
        w = lax.axis_index("s") * n_cores + lax.axis_index("c")
        sbase = w * s_w

        def xrow(t):
            return (t % batch) * seq_len + sbase + (t // batch) * _X_ROWS

        def x_copy(t, p):
            return pltpu.make_async_copy(
                x_hbm.at[pl.ds(xrow(t), _X_ROWS)],
                obuf.at[pl.ds(p * _X_ROWS, _X_ROWS)], sem_x)

        def out_copy(t, p):
            return pltpu.make_async_copy(
                obuf.at[pl.ds(p * _X_ROWS, _X_ROWS)],
                o_hbm.at[pl.ds(xrow(t), _X_ROWS)], sem_o)

        def pe_copy(c):
            return pltpu.make_async_copy(
                pe_hbm.at[pl.ds(sbase + c * _PE_ROWS, _PE_ROWS)],
                pbuf.at[pl.ds(lax.rem(c, 3) * _PE_ROWS, _PE_ROWS)], sem_p)

        x_copy(0, 0).start()
        pe_copy(0).start()
        pe_copy(1).start()

        @pl.loop(0, n_sc, step=2)
        def _step_loop(u):
            for q in range(2 * batch):
                sc = u + q // batch
                b = q % batch
                t = sc * batch + b
                p = b % 2

                if b == 0:
                    pe_copy(2 * sc).wait()
                    pe_copy(2 * sc + 1).wait()
                    @pl.when(2 * sc + 2 < n_pe)
                    def _():
                        pe_copy(2 * sc + 2).start()

                x_copy(t, p).wait()

                for h in range(2):
                    ob_h = obuf.at[pl.ds(p * _X_ROWS + h * _PE_ROWS,
                                         _PE_ROWS)]
                    poff = lax.rem(2 * sc + h, 3) * _PE_ROWS

                    @plsc.parallel_loop(0, hgroups, step=1, unroll=8)
                    def _(g):
                        r = g // gpr
                        j = (g % gpr) * _LANES
                        plsc.addupdate(ob_h.at[r, pl.ds(j, _LANES)],
                                       pbuf[poff + r, pl.ds(j, _LANES)])

                out_copy(t, p).start()

                @pl.when(t >= 1)
                def _():
                    out_copy(t - 1, (p + 1) % 2).wait()

                @pl.when(t + 1 < n_steps)
                def _():
                    x_copy(t + 1, (p + 1) % 2).start()

                if b == batch - 1:
                    @pl.when(2 * sc + 3 < n_pe)
                    def _():
                        pe_copy(2 * sc + 3).start()

        out_copy(n_steps - 1, (n_steps - 1) % 2).wait()

    return sc_add


def kernel(x, pe):
    batch, seq_len, d_model = x.shape
    n_workers = 32
    n_cores = 2

    x2 = x.reshape(batch * seq_len, d_model)
    pe2 = pe[:seq_len]
    fn = _make_sc_add(batch, seq_len, d_model, n_workers, n_cores)
    out = fn(x2, pe2)
    return out.reshape(x.shape)


# speedup vs baseline: 1.2618x; 1.2618x over previous
import functools

import jax
import jax.numpy as jnp
from jax import lax
from jax.experimental import pallas as pl
from jax.experimental.pallas import tpu as pltpu
from jax.experimental.pallas import tpu_sc as plsc

_LANES = 16
_PE_ROWS = 16
_X_ROWS = 32


@functools.cache
def _make_sc_add(batch, seq_len, d_model, n_workers, n_cores):
    n_rows = batch * seq_len
    s_w = seq_len // n_workers
    n_sc = s_w // _X_ROWS
    n_steps = n_sc * batch
    n_pe = s_w // _PE_ROWS
    hgroups = _PE_ROWS * d_model // _LANES
    gpr = d_model // _LANES

    mesh = plsc.VectorSubcoreMesh(core_axis_name="c", subcore_axis_name="s")

    @functools.partial(
        pl.kernel,
        out_type=jax.ShapeDtypeStruct((n_rows, d_model), jnp.float32),
        mesh=mesh,
        scratch_types=(
            [pltpu.VMEM((2 * _X_ROWS, d_model), jnp.float32),
             pltpu.VMEM((3 * _PE_ROWS, d_model), jnp.float32)]
            + [pltpu.SemaphoreType.DMA for _ in range(3)]
        ),
        compiler_params=pltpu.CompilerParams(use_tc_tiling_on_sc=True),
    )
    def sc_add(x_hbm, pe_hbm, o_hbm, obuf, pbuf, sem_x, sem_o, sem_p):
        w = lax.axis_index("s") * n_cores + lax.axis_index("c")
        sbase = w * s_w

        def xrow(t):
            return (t % batch) * seq_len + sbase + (t // batch) * _X_ROWS

        def x_copy(t, p):
            return pltpu.make_async_copy(
                x_hbm.at[pl.ds(xrow(t), _X_ROWS)],
                obuf.at[pl.ds(p * _X_ROWS, _X_ROWS)], sem_x)

        def out_copy(t, p):
            return pltpu.make_async_copy(
                obuf.at[pl.ds(p * _X_ROWS, _X_ROWS)],
                o_hbm.at[pl.ds(xrow(t), _X_ROWS)], sem_o)

        def pe_copy(c):
            return pltpu.make_async_copy(
                pe_hbm.at[pl.ds(sbase + c * _PE_ROWS, _PE_ROWS)],
                pbuf.at[pl.ds(lax.rem(c, 3) * _PE_ROWS, _PE_ROWS)], sem_p)

        x_copy(0, 0).start()
        pe_copy(0).start()
        pe_copy(1).start()

        @pl.loop(0, n_sc, step=2)
        def _step_loop(u):
            for q in range(2 * batch):
                sc = u + q // batch
                b = q % batch
                t = sc * batch + b
                p = b % 2

                if b == 0:
                    pe_copy(2 * sc).wait()
                    pe_copy(2 * sc + 1).wait()
                    @pl.when(2 * sc + 2 < n_pe)
                    def _():
                        pe_copy(2 * sc + 2).start()

                x_copy(t, p).wait()

                @pl.when(t >= 1)
                def _():
                    out_copy(t - 1, (p + 1) % 2).wait()

                @pl.when(t + 1 < n_steps)
                def _():
                    x_copy(t + 1, (p + 1) % 2).start()

                for h in range(2):
                    ob_h = obuf.at[pl.ds(p * _X_ROWS + h * _PE_ROWS,
                                         _PE_ROWS)]
                    poff = lax.rem(2 * sc + h, 3) * _PE_ROWS

                    @plsc.parallel_loop(0, hgroups, step=1, unroll=8)
                    def _(g):
                        r = g // gpr
                        j = (g % gpr) * _LANES
                        plsc.addupdate(ob_h.at[r, pl.ds(j, _LANES)],
                                       pbuf[poff + r, pl.ds(j, _LANES)])

                out_copy(t, p).start()

                if b == batch - 1:
                    @pl.when(2 * sc + 3 < n_pe)
                    def _():
                        pe_copy(2 * sc + 3).start()

        out_copy(n_steps - 1, (n_steps - 1) % 2).wait()

    return sc_add


def kernel(x, pe):
    batch, seq_len, d_model = x.shape
    n_workers = 32
    n_cores = 2

    x2 = x.reshape(batch * seq_len, d_model)
    pe2 = pe[:seq_len]
    fn = _make_sc_add(batch, seq_len, d_model, n_workers, n_cores)
    out = fn(x2, pe2)
    return out.reshape(x.shape)


# speedup vs baseline: 1.4639x; 1.1602x over previous
import functools

import jax
import jax.numpy as jnp
from jax import lax
from jax.experimental import pallas as pl
from jax.experimental.pallas import tpu as pltpu
from jax.experimental.pallas import tpu_sc as plsc

_LANES = 16


@functools.cache
def _make_sc_add(batch, seq_len, d_model, n_workers, n_cores, chunk_rows):
    n_rows = batch * seq_len
    s_w = seq_len // n_workers
    n_sc = s_w // chunk_rows
    n_steps = n_sc * batch
    groups = chunk_rows * d_model // _LANES
    gpr = d_model // _LANES

    mesh = plsc.VectorSubcoreMesh(core_axis_name="c", subcore_axis_name="s")

    @functools.partial(
        pl.kernel,
        out_type=jax.ShapeDtypeStruct((n_rows, d_model), jnp.float32),
        mesh=mesh,
        scratch_types=(
            [pltpu.VMEM((chunk_rows, d_model), jnp.float32) for _ in range(6)]
            + [pltpu.SemaphoreType.DMA for _ in range(10)]
        ),
        compiler_params=pltpu.CompilerParams(use_tc_tiling_on_sc=True),
    )
    def sc_add(x_hbm, pe_hbm, o_hbm,
               ob0, ob1, ob2, ob3, pb0, pb1,
               sx0, sx1, sx2, sx3, so0, so1, so2, so3, sp0, sp1):
        w = lax.axis_index("s") * n_cores + lax.axis_index("c")
        sbase = w * s_w
        obufs = (ob0, ob1, ob2, ob3)
        pbufs = (pb0, pb1)
        sin_x = (sx0, sx1, sx2, sx3)
        souts = (so0, so1, so2, so3)
        sin_p = (sp0, sp1)

        def xrow(t):
            return (t % batch) * seq_len + sbase + (t // batch) * chunk_rows

        def x_copy(t, b):
            return pltpu.make_async_copy(
                x_hbm.at[pl.ds(xrow(t), chunk_rows)], obufs[b], sin_x[b])

        def pe_copy(sc, b):
            return pltpu.make_async_copy(
                pe_hbm.at[pl.ds(sbase + sc * chunk_rows, chunk_rows)],
                pbufs[b], sin_p[b])

        def out_copy(t, b):
            return pltpu.make_async_copy(
                obufs[b], o_hbm.at[pl.ds(xrow(t), chunk_rows)], souts[b])

        for b in range(2):
            x_copy(b, b).start()
            pe_copy(b, b).start()

        @pl.loop(0, n_steps, step=2 * batch)
        def _step_loop(tt):
            for q in range(2 * batch):
                t = tt + q
                b = q % 4
                pset = (q // batch) % 2
                sc = t // batch

                if q % batch == 0:
                    pe_copy(sc, pset).wait()

                x_copy(t, b).wait()

                @pl.when(t + 2 < n_steps)
                def _():
                    @pl.when(t >= 2)
                    def _():
                        out_copy(t - 2, (q + 2) % 4).wait()
                    x_copy(t + 2, (q + 2) % 4).start()

                pb, ob = pbufs[pset], obufs[b]

                @plsc.parallel_loop(0, groups, step=1, unroll=8)
                def _(g):
                    r = g // gpr
                    j = (g % gpr) * _LANES
                    plsc.addupdate(ob.at[r, pl.ds(j, _LANES)],
                                   pb[r, pl.ds(j, _LANES)])

                out_copy(t, b).start()

                if q % batch == batch - 1:
                    @pl.when(sc + 2 < n_sc)
                    def _():
                        pe_copy(sc + 2, pset).start()

        for d in range(4):
            t = n_steps - 4 + d
            out_copy(t, t % 4).wait()

    return sc_add


def kernel(x, pe):
    batch, seq_len, d_model = x.shape
    n_workers = 32
    n_cores = 2
    chunk_rows = 16

    x2 = x.reshape(batch * seq_len, d_model)
    pe2 = pe[:seq_len]
    fn = _make_sc_add(batch, seq_len, d_model, n_workers, n_cores,
                      chunk_rows)
    out = fn(x2, pe2)
    return out.reshape(x.shape)


# speedup vs baseline: 1.4653x; 1.0010x over previous
import functools

import jax
import jax.numpy as jnp
from jax import lax
from jax.experimental import pallas as pl
from jax.experimental.pallas import tpu as pltpu
from jax.experimental.pallas import tpu_sc as plsc

_LANES = 16


@functools.cache
def _make_sc_add(batch, seq_len, d_model, n_workers, n_cores, chunk_rows):
    n_rows = batch * seq_len
    s_w = seq_len // n_workers
    n_sc = s_w // chunk_rows
    n_steps = n_sc * batch
    groups = chunk_rows * d_model // _LANES
    gpr = d_model // _LANES

    mesh = plsc.VectorSubcoreMesh(core_axis_name="c", subcore_axis_name="s")

    @functools.partial(
        pl.kernel,
        out_type=jax.ShapeDtypeStruct((n_rows, d_model), jnp.float32),
        mesh=mesh,
        scratch_types=(
            [pltpu.VMEM((chunk_rows, d_model), jnp.float32) for _ in range(6)]
            + [pltpu.SemaphoreType.DMA for _ in range(10)]
        ),
        compiler_params=pltpu.CompilerParams(use_tc_tiling_on_sc=True,
                                             skip_device_barrier=True),
    )
    def sc_add(x_hbm, pe_hbm, o_hbm,
               ob0, ob1, ob2, ob3, pb0, pb1,
               sx0, sx1, sx2, sx3, so0, so1, so2, so3, sp0, sp1):
        w = lax.axis_index("s") * n_cores + lax.axis_index("c")
        sbase = w * s_w
        obufs = (ob0, ob1, ob2, ob3)
        pbufs = (pb0, pb1)
        sin_x = (sx0, sx1, sx2, sx3)
        souts = (so0, so1, so2, so3)
        sin_p = (sp0, sp1)

        def xrow(t):
            return (t % batch) * seq_len + sbase + (t // batch) * chunk_rows

        def x_copy(t, b):
            return pltpu.make_async_copy(
                x_hbm.at[pl.ds(xrow(t), chunk_rows)], obufs[b], sin_x[b])

        def pe_copy(sc, b):
            return pltpu.make_async_copy(
                pe_hbm.at[pl.ds(sbase + sc * chunk_rows, chunk_rows)],
                pbufs[b], sin_p[b])

        def out_copy(t, b):
            return pltpu.make_async_copy(
                obufs[b], o_hbm.at[pl.ds(xrow(t), chunk_rows)], souts[b])

        for b in range(2):
            x_copy(b, b).start()
            pe_copy(b, b).start()

        @pl.loop(0, n_steps, step=2 * batch)
        def _step_loop(tt):
            for q in range(2 * batch):
                t = tt + q
                b = q % 4
                pset = (q // batch) % 2
                sc = t // batch

                if q % batch == 0:
                    pe_copy(sc, pset).wait()

                x_copy(t, b).wait()

                @pl.when(t + 2 < n_steps)
                def _():
                    @pl.when(t >= 2)
                    def _():
                        out_copy(t - 2, (q + 2) % 4).wait()
                    x_copy(t + 2, (q + 2) % 4).start()

                pb, ob = pbufs[pset], obufs[b]

                @plsc.parallel_loop(0, groups, step=1, unroll=8)
                def _(g):
                    r = g // gpr
                    j = (g % gpr) * _LANES
                    plsc.addupdate(ob.at[r, pl.ds(j, _LANES)],
                                   pb[r, pl.ds(j, _LANES)])

                out_copy(t, b).start()

                if q % batch == batch - 1:
                    @pl.when(sc + 2 < n_sc)
                    def _():
                        pe_copy(sc + 2, pset).start()

        for d in range(4):
            t = n_steps - 4 + d
            out_copy(t, t % 4).wait()

    return sc_add


def kernel(x, pe):
    batch, seq_len, d_model = x.shape
    n_workers = 32
    n_cores = 2
    chunk_rows = 16

    x2 = x.reshape(batch * seq_len, d_model)
    pe2 = pe[:seq_len]
    fn = _make_sc_add(batch, seq_len, d_model, n_workers, n_cores,
                      chunk_rows)
    out = fn(x2, pe2)
    return out.reshape(x.shape)
